# Initial kernel scaffold; baseline (speedup 1.0000x reference)
import jax
import jax.numpy as jnp
from jax.experimental import pallas as pl


def kernel(x, edge_index, batch, prior_logits, emit_logits, gate_W1, gate_b1, gate_W2, gate_b2, contrastive, out_W, out_b):
    raise NotImplementedError("write your pallas kernel here")


# speedup vs baseline: 56.5416x; 56.5416x over previous
import functools

import jax
import jax.numpy as jnp
from jax import lax
from jax.experimental import pallas as pl
from jax.experimental.pallas import tpu as pltpu
from jax.experimental.pallas import tpu_sc as plsc

N_NODES = 100000
N_GEN = 16
C = 8
M = 32
GATE = 32
N_GRAPHS = 512
OUT = 64
PAIRS = N_GEN * (N_GEN - 1) // 2

NC = 2
NS = 16
NW = NC * NS
N_PAD = 102400
CHUNK = N_PAD // NW
BINS = (N_GRAPHS + 1) * M
LANES = 16


def _sc_hist(x_hbm, b_hbm, zeros_hbm, out_hbm, x_v, b_v, hist_v):
    wid = lax.axis_index("s") * NC + lax.axis_index("c")
    base = wid * CHUNK
    pltpu.sync_copy(zeros_hbm, hist_v)
    pltpu.sync_copy(x_hbm.at[pl.ds(base, CHUNK)], x_v)
    pltpu.sync_copy(b_hbm.at[pl.ds(base, CHUNK)], b_v)
    ones = jnp.full((LANES,), 1.0, jnp.float32)

    def body(i, carry):
        xv = x_v[pl.ds(i * LANES, LANES)]
        bv = b_v[pl.ds(i * LANES, LANES)]
        k = bv * M + xv
        plsc.addupdate_scatter(hist_v, [k], ones)
        return carry

    lax.fori_loop(0, CHUNK // LANES, body, 0)
    pltpu.sync_copy(hist_v, out_hbm.at[wid])


@functools.cache
def _sc_hist_kernel():
    return pl.kernel(
        _sc_hist,
        mesh=plsc.VectorSubcoreMesh(core_axis_name="c", subcore_axis_name="s"),
        compiler_params=pltpu.CompilerParams(needs_layout_passes=False),
        out_type=jax.ShapeDtypeStruct((NW, BINS), jnp.float32),
        scratch_types=[
            pltpu.VMEM((CHUNK,), jnp.int32),
            pltpu.VMEM((CHUNK,), jnp.int32),
            pltpu.VMEM((BINS,), jnp.float32),
        ],
    )


def _tc_body(hist_ref, prior_ref, emit_ref, w1_ref, b1_ref, w2_ref, b2_ref,
             con_ref, ow_ref, ob_ref, out_ref):
    hist = jnp.sum(hist_ref[...], axis=0)
    lp = jax.nn.log_softmax(prior_ref[...], axis=1)
    le = jax.nn.log_softmax(emit_ref[...], axis=2)
    t = lp[:, :, None] + le
    tm = jnp.max(t, axis=1, keepdims=True)
    llG = jnp.log(jnp.sum(jnp.exp(t - tm), axis=1)) + tm[:, 0, :]
    def rbf(v):
        bits = lax.bitcast_convert_type(v, jnp.uint32)
        lsb = lax.shift_right_logical(bits, jnp.uint32(16)) & jnp.uint32(1)
        bits = (bits + jnp.uint32(0x7FFF) + lsb) & jnp.uint32(0xFFFF0000)
        return lax.bitcast_convert_type(bits, jnp.float32)

    bdot = lambda a, b, dims: lax.dot_general(
        rbf(a), rbf(b), dims, preferred_element_type=jnp.float32)
    h = jnp.tanh(bdot(llG, w1_ref[...], (((0,), (0,)), ((), ())))
                 + b1_ref[...])
    gate_row = bdot(w2_ref[...], h, (((0,), (1,)), ((), ()))) + b2_ref[...]
    present = hist > 0.0
    gmax = jnp.max(jnp.where(present, gate_row, -1e30), axis=1, keepdims=True)
    gmax = jnp.where(gmax < -1e29, 0.0, gmax)
    w = hist * jnp.exp(gate_row - gmax)
    denom = jnp.sum(w, axis=1, keepdims=True)
    num = jnp.sum(w[:, None, :] * llG[None, :, :], axis=2)
    r = num / (denom + 1e-16)
    std = (((1,), (0,)), ((), ()))
    c = jnp.tanh(bdot(r, con_ref[...], std))
    out_ref[...] = bdot(c, ow_ref[...], std) + ob_ref[...]


def kernel(x, edge_index, batch, prior_logits, emit_logits, gate_W1, gate_b1,
           gate_W2, gate_b2, contrastive, out_W, out_b):
    del edge_index
    pad = N_PAD - N_NODES
    x_p = jnp.concatenate([x.astype(jnp.int32), jnp.zeros((pad,), jnp.int32)])
    b_p = jnp.concatenate([batch.astype(jnp.int32),
                           jnp.full((pad,), N_GRAPHS, jnp.int32)])
    zeros = jnp.zeros((BINS,), jnp.float32)
    hist_parts = _sc_hist_kernel()(x_p, b_p, zeros)
    hist3 = hist_parts.reshape(NW, N_GRAPHS + 1, M)[:, :N_GRAPHS, :]

    out = pl.pallas_call(
        _tc_body,
        out_shape=jax.ShapeDtypeStruct((N_GRAPHS, OUT), jnp.float32),
    )(hist3, prior_logits, emit_logits, gate_W1,
      gate_b1.reshape(1, GATE), gate_W2, gate_b2.reshape(1, 1),
      contrastive, out_W, out_b.reshape(1, OUT))
    return out
